# Initial kernel scaffold; baseline (speedup 1.0000x reference)
import jax
import jax.numpy as jnp
from jax.experimental import pallas as pl


def kernel(x, edge_index, edge_weight, W1, b1, Wh, bh, W2, b2):
    raise NotImplementedError("write your pallas kernel here")


# speedup vs baseline: 8.9400x; 8.9400x over previous
import functools

import jax
import jax.numpy as jnp
from jax import lax
from jax.experimental import pallas as pl
from jax.experimental.pallas import tpu as pltpu
from jax.experimental.pallas import tpu_sc as plsc

_N_TILES_PER_SC = 16
_N_SC = 2
_N_WORKERS = _N_SC * _N_TILES_PER_SC
_C = 80
_LANES = 16


def _ceil_to(v, m):
    return (v + m - 1) // m * m


def _deg_body(n_pad, cpt, packed, ewc, degp, ebuf, wbuf, degbuf, tmp, acc, slab):
    cid = lax.axis_index("c")
    sid = lax.axis_index("s")
    t = cid * _N_TILES_PER_SC + sid
    per = n_pad // _N_TILES_PER_SC
    zero16 = jnp.zeros((_LANES,), jnp.float32)

    def zbody(i, _):
        degbuf[pl.ds(i * _LANES, _LANES)] = zero16
        return 0

    lax.fori_loop(0, n_pad // _LANES, zbody, 0)

    base = t * cpt

    def cbody(i, _):
        pltpu.sync_copy(packed.at[base + i], ebuf)
        pltpu.sync_copy(ewc.at[base + i], wbuf)
        for j in range(_C // _LANES):
            sl = pl.ds(j * _LANES, _LANES)
            col_v = ebuf[1, sl]
            ew_v = wbuf[sl]
            plsc.addupdate_scatter(degbuf, [col_v], ew_v)
        return 0

    lax.fori_loop(0, cpt, cbody, 0)

    pltpu.sync_copy(degbuf, slab.at[sid])
    plsc.subcore_barrier()
    for s in range(_N_TILES_PER_SC):
        pltpu.sync_copy(slab.at[s, pl.ds(sid * per, per)], tmp)
        for k in range(per // _LANES):
            sl = pl.ds(k * _LANES, _LANES)
            if s == 0:
                acc[sl] = tmp[sl]
            else:
                acc[sl] = acc[sl] + tmp[sl]
    pltpu.sync_copy(acc, degp.at[cid, pl.ds(sid * per, per)])


def _agg_body(n_pad, cpt, h, packed, ewc, dis, out_p,
              ebuf, wbuf, rows, disbuf, nbuf, zbuf, acc_sh, sem):
    cid = lax.axis_index("c")
    sid = lax.axis_index("s")
    t = cid * _N_TILES_PER_SC + sid
    per = n_pad // _N_TILES_PER_SC
    zero16 = jnp.zeros((_LANES,), jnp.float32)

    pltpu.sync_copy(dis, disbuf)

    for r in range(32):
        for q in range(8):
            zbuf[r, pl.ds(q * _LANES, _LANES)] = zero16
    for k in range(per // 32):
        pltpu.sync_copy(zbuf, acc_sh.at[pl.ds(sid * per + k * 32, 32)])
    plsc.subcore_barrier()

    base = t * cpt

    def cbody(i, _):
        pltpu.sync_copy(packed.at[base + i], ebuf)
        pltpu.sync_copy(ewc.at[base + i], wbuf)
        pltpu.async_copy(h.at[ebuf.at[0]], rows, sem).wait()
        for j in range(_C // _LANES):
            sl = pl.ds(j * _LANES, _LANES)
            row_v = ebuf[0, sl]
            col_v = ebuf[1, sl]
            ew_v = wbuf[sl]
            n_v = (plsc.load_gather(disbuf, [row_v]) * ew_v
                   * plsc.load_gather(disbuf, [col_v]))
            nbuf[sl] = n_v
        for e in range(_C):
            nb = plsc.load_gather(nbuf, [jnp.full((_LANES,), e, jnp.int32)])
            for q in range(8):
                sl = pl.ds(q * _LANES, _LANES)
                rows[e, sl] = rows[e, sl] * nb
        pltpu.sync_copy(rows, acc_sh.at[ebuf.at[1]], add=True)
        return 0

    lax.fori_loop(0, cpt, cbody, 0)

    plsc.subcore_barrier()
    for k in range(per // 32):
        sl = pl.ds(sid * per + k * 32, 32)
        pltpu.sync_copy(acc_sh.at[sl], out_p.at[cid, sl])


def _dis_tc(dp_ref, o_ref):
    d = dp_ref[0] + dp_ref[1]
    o_ref[...] = lax.rsqrt(jnp.maximum(d, 1e-12))


def _mm_tc(x_ref, w_ref, o_ref):
    o_ref[...] = jnp.dot(x_ref[...], w_ref[...],
                         preferred_element_type=jnp.float32)


def _hidden_tc(p0_ref, p1_ref, b1_ref, wh_ref, bh_ref, w2_ref, o_ref):
    a = jnp.maximum(p0_ref[...] + p1_ref[...] + b1_ref[...], 0.0)
    tt = jnp.dot(a, wh_ref[...], preferred_element_type=jnp.float32)
    tt = jnp.maximum(tt + bh_ref[...], 0.0)
    o_ref[...] = jnp.dot(tt, w2_ref[...], preferred_element_type=jnp.float32)


def _logsoftmax_tc(p0_ref, p1_ref, b2_ref, o_ref):
    s = p0_ref[...] + p1_ref[...] + b2_ref[...]
    m = jnp.max(s, axis=1, keepdims=True)
    ex = jnp.exp(s - m)
    lse = jnp.log(jnp.sum(ex, axis=1, keepdims=True))
    o_ref[...] = s - m - lse


def kernel(x, edge_index, edge_weight, W1, b1, Wh, bh, W2, b2):
    n, d_in = x.shape
    d_h = Wh.shape[0]
    d_out = W2.shape[1]
    e = edge_index.shape[1]
    f32 = jnp.float32

    n_pad = _ceil_to(n, _N_TILES_PER_SC * 32)
    et = e + n
    n_chunks = _ceil_to(et, _C * _N_WORKERS) // _C
    cpt = n_chunks // _N_WORKERS
    pad = n_chunks * _C - et

    loop_idx = jnp.arange(n, dtype=jnp.int32)
    zpad = jnp.zeros((pad,), jnp.int32)
    rows_all = jnp.concatenate([edge_index[0], loop_idx, zpad])
    cols_all = jnp.concatenate([edge_index[1], loop_idx, zpad])
    ew_all = jnp.concatenate([edge_weight.astype(f32), jnp.ones((n,), f32),
                              jnp.zeros((pad,), f32)])
    packed = jnp.stack([rows_all, cols_all], axis=0)
    packed = packed.reshape(2, n_chunks, _C).transpose(1, 0, 2)
    ewc = ew_all.reshape(n_chunks, _C)

    mesh = plsc.VectorSubcoreMesh(core_axis_name="c", subcore_axis_name="s")
    per = n_pad // _N_TILES_PER_SC

    deg_call = pl.kernel(
        functools.partial(_deg_body, n_pad, cpt),
        out_type=jax.ShapeDtypeStruct((_N_SC, n_pad), f32),
        mesh=mesh,
        scratch_types=[
            pltpu.VMEM((2, _C), jnp.int32),
            pltpu.VMEM((_C,), f32),
            pltpu.VMEM((n_pad,), f32),
            pltpu.VMEM((per,), f32),
            pltpu.VMEM((per,), f32),
            pltpu.VMEM_SHARED((_N_TILES_PER_SC, n_pad), f32),
        ],
        compiler_params=pltpu.CompilerParams(needs_layout_passes=False),
    )
    degp = deg_call(packed, ewc)

    dis = pl.pallas_call(
        _dis_tc,
        out_shape=jax.ShapeDtypeStruct((n_pad // 128, 128), f32),
    )(degp.reshape(_N_SC, n_pad // 128, 128))
    dis = dis.reshape(n_pad)

    blk = 1000
    grid = n // blk
    h1 = pl.pallas_call(
        _mm_tc,
        grid=(grid,),
        in_specs=[
            pl.BlockSpec((blk, d_in), lambda i: (i, 0)),
            pl.BlockSpec((d_in, d_h), lambda i: (0, 0)),
        ],
        out_specs=pl.BlockSpec((blk, d_h), lambda i: (i, 0)),
        out_shape=jax.ShapeDtypeStruct((n, d_h), f32),
    )(x, W1)

    agg_call = pl.kernel(
        functools.partial(_agg_body, n_pad, cpt),
        out_type=jax.ShapeDtypeStruct((_N_SC, n_pad, d_h), f32),
        mesh=mesh,
        scratch_types=[
            pltpu.VMEM((2, _C), jnp.int32),
            pltpu.VMEM((_C,), f32),
            pltpu.VMEM((_C, d_h), f32),
            pltpu.VMEM((n_pad,), f32),
            pltpu.VMEM((_C,), f32),
            pltpu.VMEM((32, d_h), f32),
            pltpu.VMEM_SHARED((n_pad, d_h), f32),
            pltpu.SemaphoreType.DMA,
        ],
        compiler_params=pltpu.CompilerParams(needs_layout_passes=False),
    )

    p = agg_call(h1, packed, ewc, dis)

    g = pl.pallas_call(
        _hidden_tc,
        grid=(grid,),
        in_specs=[
            pl.BlockSpec((blk, d_h), lambda i: (i, 0)),
            pl.BlockSpec((blk, d_h), lambda i: (i, 0)),
            pl.BlockSpec((1, d_h), lambda i: (0, 0)),
            pl.BlockSpec((d_h, d_h), lambda i: (0, 0)),
            pl.BlockSpec((1, d_h), lambda i: (0, 0)),
            pl.BlockSpec((d_h, d_out), lambda i: (0, 0)),
        ],
        out_specs=pl.BlockSpec((blk, d_out), lambda i: (i, 0)),
        out_shape=jax.ShapeDtypeStruct((n, d_out), f32),
    )(p[0, :n], p[1, :n], b1.reshape(1, d_h), Wh, bh.reshape(1, d_h), W2)

    p2 = agg_call(g, packed, ewc, dis)

    out = pl.pallas_call(
        _logsoftmax_tc,
        grid=(grid,),
        in_specs=[
            pl.BlockSpec((blk, d_out), lambda i: (i, 0)),
            pl.BlockSpec((blk, d_out), lambda i: (i, 0)),
            pl.BlockSpec((1, d_out), lambda i: (0, 0)),
        ],
        out_specs=pl.BlockSpec((blk, d_out), lambda i: (i, 0)),
        out_shape=jax.ShapeDtypeStruct((n, d_out), f32),
    )(p2[0, :n], p2[1, :n], b2.reshape(1, d_out))
    return out
